# baseline (speedup 1.0000x reference)
import functools

import jax
import jax.numpy as jnp
from jax import lax
from jax.experimental import pallas as pl
from jax.experimental.pallas import tpu as pltpu
from jax.experimental.pallas import tpu_sc as plsc

HD = 128
NJ = HD // 16
NT, ND, NH, NW = 16, 16, 32, 32
DOCT = 8
HCH = 4
NCH = NH // HCH
NCHUNK = DOCT * NCH
NBUF = 4

_MESH = plsc.VectorSubcoreMesh(core_axis_name="c", subcore_axis_name="s")


def _pos_body(t_hbm, d_hbm, h_hbm, w_hbm, out_hbm,
              t_v, d_v, h_v, w_v, base_v, bufs, sems):
    wid = lax.axis_index("s") * 2 + lax.axis_index("c")
    t_idx = wid // 2
    d0 = (wid % 2) * DOCT

    stage = [
        pltpu.make_async_copy(t_hbm.at[pl.ds(t_idx, 1)], t_v, sems[0]),
        pltpu.make_async_copy(d_hbm.at[pl.ds(d0, DOCT)], d_v, sems[1]),
        pltpu.make_async_copy(h_hbm.at[pl.ds(0, NH)], h_v, sems[2]),
        pltpu.make_async_copy(w_hbm.at[pl.ds(0, NW)], w_v, sems[3]),
    ]
    for cp in stage:
        cp.start()
    for cp in stage:
        cp.wait()

    for dp in range(DOCT):
        td = [t_v[0, pl.ds(16 * j, 16)] + d_v[dp, pl.ds(16 * j, 16)]
              for j in range(NJ)]

        @plsc.parallel_loop(0, NH, 1, unroll=2)
        def _h_body(h, _dp=dp, _td=td):
            for j in range(NJ):
                base_v[_dp, h, pl.ds(16 * j, 16)] = (
                    h_v[h, pl.ds(16 * j, 16)] + _td[j])

    def chunk(c, buf):
        dp = c // NCH
        hq = c % NCH
        for hl in range(HCH):
            h = hq * HCH + hl
            b = [base_v[dp, h, pl.ds(16 * j, 16)] for j in range(NJ)]

            @plsc.parallel_loop(0, NW, 1, unroll=4)
            def _w_body(w, _hl=hl, _b=b):
                for j in range(NJ):
                    buf[_hl, w, pl.ds(16 * j, 16)] = (
                        w_v[w, pl.ds(16 * j, 16)] + _b[j])

    def start_copy(c, par):
        dp = c // NCH
        hq = c % NCH
        pltpu.make_async_copy(
            bufs[par],
            out_hbm.at[t_idx, d0 + dp, pl.ds(hq * HCH, HCH)],
            sems[par]).start()

    def wait_par(par):
        pltpu.make_async_copy(
            bufs[par], out_hbm.at[0, 0, pl.ds(0, HCH)], sems[par]).wait()

    for par in range(NBUF):
        chunk(par, bufs[par])
        start_copy(par, par)

    def ring_body(p, carry):
        c0 = NBUF * p
        for par in range(NBUF):
            c = c0 + par
            wait_par(par)
            chunk(c, bufs[par])
            start_copy(c, par)
        return carry

    lax.fori_loop(1, NCHUNK // NBUF, ring_body, 0)
    for par in range(NBUF):
        wait_par(par)


@functools.partial(
    pl.kernel,
    mesh=_MESH,
    out_type=jax.ShapeDtypeStruct((NT, ND, NH, NW, HD), jnp.float32),
    scratch_types=[
        pltpu.VMEM((1, HD), jnp.float32),
        pltpu.VMEM((DOCT, HD), jnp.float32),
        pltpu.VMEM((NH, HD), jnp.float32),
        pltpu.VMEM((NW, HD), jnp.float32),
        pltpu.VMEM((DOCT, NH, HD), jnp.float32),
        [pltpu.VMEM((HCH, NW, HD), jnp.float32) for _ in range(NBUF)],
        [pltpu.SemaphoreType.DMA for _ in range(NBUF)],
    ],
)
def _pos_kernel(t_hbm, d_hbm, h_hbm, w_hbm, out_hbm, *scratch):
    _pos_body(t_hbm, d_hbm, h_hbm, w_hbm, out_hbm, *scratch)


def kernel(T, n_d, n_h, n_w, t_embed, d_embed, h_embed, w_embed):
    del T, n_d, n_h, n_w
    return _pos_kernel(t_embed, d_embed, h_embed, w_embed)


# speedup vs baseline: 1.1386x; 1.1386x over previous
import functools

import jax
import jax.numpy as jnp
from jax import lax
from jax.experimental import pallas as pl
from jax.experimental.pallas import tpu as pltpu
from jax.experimental.pallas import tpu_sc as plsc

HD = 128
NJ = HD // 16
NT, ND, NH, NW = 16, 16, 32, 32
DOCT = 8
HCH = 8
NCH = NH // HCH
NCHUNK = DOCT * NCH
NBUF = 2

_MESH = plsc.VectorSubcoreMesh(core_axis_name="c", subcore_axis_name="s")


def _pos_body(t_hbm, d_hbm, h_hbm, w_hbm, out_hbm,
              t_v, d_v, h_v, w_v, base_v, bufs, sems):
    wid = lax.axis_index("s") * 2 + lax.axis_index("c")
    t_idx = wid // 2
    d0 = (wid % 2) * DOCT

    stage = [
        pltpu.make_async_copy(t_hbm.at[pl.ds(t_idx, 1)], t_v, sems[0]),
        pltpu.make_async_copy(d_hbm.at[pl.ds(d0, DOCT)], d_v, sems[1]),
        pltpu.make_async_copy(h_hbm.at[pl.ds(0, NH)], h_v, sems[2]),
        pltpu.make_async_copy(w_hbm.at[pl.ds(0, NW)], w_v, sems[3]),
    ]
    for cp in stage:
        cp.start()
    for cp in stage:
        cp.wait()

    for dp in range(DOCT):
        td = [t_v[0, pl.ds(16 * j, 16)] + d_v[dp, pl.ds(16 * j, 16)]
              for j in range(NJ)]

        @plsc.parallel_loop(0, NH, 1, unroll=2)
        def _h_body(h, _dp=dp, _td=td):
            for j in range(NJ):
                base_v[_dp, h, pl.ds(16 * j, 16)] = (
                    h_v[h, pl.ds(16 * j, 16)] + _td[j])

    def chunk(c, buf):
        dp = c // NCH
        hq = c % NCH
        for hl in range(HCH):
            h = hq * HCH + hl
            b = [base_v[dp, h, pl.ds(16 * j, 16)] for j in range(NJ)]

            @plsc.parallel_loop(0, NW, 1, unroll=4)
            def _w_body(w, _hl=hl, _b=b):
                for j in range(NJ):
                    buf[_hl, w, pl.ds(16 * j, 16)] = (
                        w_v[w, pl.ds(16 * j, 16)] + _b[j])

    def start_copy(c, par):
        dp = c // NCH
        hq = c % NCH
        pltpu.make_async_copy(
            bufs[par],
            out_hbm.at[t_idx, d0 + dp, pl.ds(hq * HCH, HCH)],
            sems[par]).start()

    def wait_par(par):
        pltpu.make_async_copy(
            bufs[par], out_hbm.at[0, 0, pl.ds(0, HCH)], sems[par]).wait()

    for par in range(NBUF):
        chunk(par, bufs[par])
        start_copy(par, par)

    def ring_body(p, carry):
        c0 = NBUF * p
        for par in range(NBUF):
            c = c0 + par
            wait_par(par)
            chunk(c, bufs[par])
            start_copy(c, par)
        return carry

    lax.fori_loop(1, NCHUNK // NBUF, ring_body, 0)
    for par in range(NBUF):
        wait_par(par)


@functools.partial(
    pl.kernel,
    mesh=_MESH,
    out_type=jax.ShapeDtypeStruct((NT, ND, NH, NW, HD), jnp.float32),
    scratch_types=[
        pltpu.VMEM((1, HD), jnp.float32),
        pltpu.VMEM((DOCT, HD), jnp.float32),
        pltpu.VMEM((NH, HD), jnp.float32),
        pltpu.VMEM((NW, HD), jnp.float32),
        pltpu.VMEM((DOCT, NH, HD), jnp.float32),
        [pltpu.VMEM((HCH, NW, HD), jnp.float32) for _ in range(NBUF)],
        [pltpu.SemaphoreType.DMA for _ in range(max(NBUF, 4))],
    ],
)
def _pos_kernel(t_hbm, d_hbm, h_hbm, w_hbm, out_hbm, *scratch):
    _pos_body(t_hbm, d_hbm, h_hbm, w_hbm, out_hbm, *scratch)


def kernel(T, n_d, n_h, n_w, t_embed, d_embed, h_embed, w_embed):
    del T, n_d, n_h, n_w
    return _pos_kernel(t_embed, d_embed, h_embed, w_embed)
